# baseline (speedup 1.0000x reference)
import functools

import jax
import jax.numpy as jnp
from jax import lax
from jax.experimental import pallas as pl
from jax.experimental.pallas import tpu as pltpu

_TB = 128


def _body(xs_ref, w1_ref, w2_ref, fw1_ref, fw2_ref, out_ref,
          x1, o1, p, o2, flat, h):
    TB = _TB
    f32 = jnp.float32
    bf16 = jnp.bfloat16

    ii = lax.broadcasted_iota(jnp.int32, (8, TB), 0)
    probe = ii.astype(f32)
    expect_fwd = jnp.where(ii == 0, 7, ii - 1).astype(f32)
    roll_is_fwd = (pltpu.roll(probe, 1, 0) == expect_fwd)[None]

    def roll_sub(v, shift):
        s = shift % 8
        fwd = pltpu.roll(v, s, 1)
        bwd = pltpu.roll(v, (8 - s) % 8, 1)
        return jnp.where(roll_is_fwd, fwd, bwd)

    ones_slab = jnp.ones((8, TB), bf16)

    for hh in range(28):
        cs = slice(hh * TB, (hh + 1) * TB)
        for di in range(3):
            for t in range(6):
                r0 = (di * 6 + t) * 8
                x1[r0:r0 + 8, cs] = xs_ref[t, hh + di]
        x1[144:152, cs] = ones_slab

    o1[...] = jnp.dot(w1_ref[...], x1[...], preferred_element_type=f32)

    p[:192, 0:TB] = jnp.zeros((192, TB), bf16)
    p[:192, 15 * TB:16 * TB] = jnp.zeros((192, TB), bf16)
    p[192:200, :] = jnp.ones((8, 16 * TB), bf16)

    jj = lax.broadcasted_iota(jnp.int32, (6, 8, TB), 1)
    mask7 = jj < 7

    for r in range(14):
        a = o1[:, (2 * r) * TB:(2 * r + 1) * TB]
        b = o1[:, (2 * r + 1) * TB:(2 * r + 2) * TB]
        m = jnp.maximum(a, b).reshape(6, 4, 8, TB)
        q0 = jnp.maximum(m[:, 0], m[:, 1])
        q1 = jnp.maximum(m[:, 2], m[:, 3])
        q0 = jnp.where(mask7, jnp.maximum(q0, 0.0), 0.0)
        q1 = jnp.where(mask7, jnp.maximum(q1, 0.0), 0.0)
        t0 = roll_sub(q1, 1)
        t3 = jnp.where(mask7, roll_sub(q0, -1), 0.0)
        cs = slice((r + 1) * TB, (r + 2) * TB)
        p[0:48, cs] = t0.reshape(48, TB).astype(bf16)
        p[48:96, cs] = q0.reshape(48, TB).astype(bf16)
        p[96:144, cs] = q1.reshape(48, TB).astype(bf16)
        p[144:192, cs] = t3.reshape(48, TB).astype(bf16)

    o2[...] = jnp.dot(w2_ref[0], p[:, 0:14 * TB], preferred_element_type=f32)
    o2[...] += jnp.dot(w2_ref[1], p[:, TB:15 * TB], preferred_element_type=f32)
    o2[...] += jnp.dot(w2_ref[2], p[:, 2 * TB:16 * TB], preferred_element_type=f32)

    for rp in range(7):
        a = o2[:, (2 * rp) * TB:(2 * rp + 1) * TB]
        b = o2[:, (2 * rp + 1) * TB:(2 * rp + 2) * TB]
        m = jnp.maximum(a, b).reshape(6, 2, 8, TB)
        f = jnp.maximum(jnp.maximum(m[:, 0], m[:, 1]), 0.0).astype(bf16)
        for c in range(6):
            r0 = c * 56 + rp * 8
            flat[r0:r0 + 8, :] = f[c]
    flat[336:344, :] = ones_slab

    hv = jnp.dot(fw1_ref[...], flat[...], preferred_element_type=f32)
    h[0:64, :] = jnp.maximum(hv, 0.0).astype(bf16)
    h[64:72, :] = ones_slab
    out_ref[...] = jnp.dot(fw2_ref[...], h[...], preferred_element_type=f32)


@jax.jit
def _forward(x, w1, b1, w2, b2, fw1, fb1, fw2, fb2):
    B = x.shape[0]
    TB = _TB
    n_tiles = pl.cdiv(B, TB)
    B_pad = n_tiles * TB
    f32 = jnp.float32
    bf16 = jnp.bfloat16

    xb = x.reshape(B, 784).astype(bf16)
    if B_pad != B:
        xb = jnp.pad(xb, ((0, B_pad - B), (0, 0)))
    xT = jnp.transpose(xb)
    xr = xT.reshape(28, 28, B_pad)
    xp = jnp.pad(xr, ((1, 1), (1, 7), (0, 0)))
    xs = jnp.stack([xp[:, t:t + 32:4, :] for t in range(6)], axis=0)

    w1r = w1.reshape(6, 3, 3).astype(f32)
    A = jnp.zeros((6, 4, 3, 6), f32)
    for p_ in range(4):
        for dj in range(3):
            A = A.at[:, p_, :, p_ + dj].set(w1r[:, :, dj])
    eye8 = jnp.eye(8, dtype=f32)
    W1 = jnp.einsum('cpdt,jk->cpjdtk', A, eye8).reshape(192, 144)
    W1f = jnp.zeros((192, 152), f32).at[:, :144].set(W1)
    W1f = W1f.at[:, 144].set(jnp.repeat(b1.astype(f32), 32)).astype(bf16)

    Bm = jnp.zeros((3, 6, 2, 4, 6), f32)
    for di in range(3):
        for q in range(2):
            for dj in range(3):
                Bm = Bm.at[di, :, q, q + dj, :].set(w2[:, :, di, dj].astype(f32))
    W2 = jnp.einsum('dcqti,jk->dcqjtik', Bm, eye8).reshape(3, 96, 192)
    W2f = jnp.zeros((3, 96, 200), f32).at[:, :, :192].set(W2)
    W2f = W2f.at[1, :, 192].set(jnp.repeat(b2.astype(f32), 16)).astype(bf16)

    fw1r = fw1.reshape(64, 6, 7, 7).astype(f32)
    fw1p = jnp.pad(fw1r, ((0, 0), (0, 0), (0, 0), (0, 1))).reshape(64, 336)
    FW1 = jnp.zeros((64, 344), f32).at[:, :336].set(fw1p)
    FW1 = FW1.at[:, 336].set(fb1.astype(f32)).astype(bf16)
    FW2 = jnp.zeros((10, 72), f32).at[:, :64].set(fw2.astype(f32))
    FW2 = FW2.at[:, 64].set(fb2.astype(f32)).astype(bf16)

    out = pl.pallas_call(
        _body,
        out_shape=jax.ShapeDtypeStruct((10, B_pad), f32),
        grid=(n_tiles,),
        in_specs=[
            pl.BlockSpec((6, 30, 8, TB), lambda i: (0, 0, 0, i)),
            pl.BlockSpec((192, 152), lambda i: (0, 0)),
            pl.BlockSpec((3, 96, 200), lambda i: (0, 0, 0)),
            pl.BlockSpec((64, 344), lambda i: (0, 0)),
            pl.BlockSpec((10, 72), lambda i: (0, 0)),
        ],
        out_specs=pl.BlockSpec((10, TB), lambda i: (0, i)),
        scratch_shapes=[
            pltpu.VMEM((152, 28 * TB), bf16),
            pltpu.VMEM((192, 28 * TB), f32),
            pltpu.VMEM((200, 16 * TB), bf16),
            pltpu.VMEM((96, 14 * TB), f32),
            pltpu.VMEM((344, TB), bf16),
            pltpu.VMEM((72, TB), bf16),
        ],
        compiler_params=pltpu.CompilerParams(
            dimension_semantics=("parallel",),
            vmem_limit_bytes=32 * 1024 * 1024,
        ),
    )(xs, W1f, W2f, FW1, FW2)

    return jnp.transpose(out[:, :B])


def kernel(x, w1, b1, w2, b2, fw1, fb1, fw2, fb2):
    return _forward(x, w1, b1, w2, b2, fw1, fb1, fw2, fb2)


# speedup vs baseline: 1.1934x; 1.1934x over previous
import functools

import jax
import jax.numpy as jnp
from jax import lax
from jax.experimental import pallas as pl
from jax.experimental.pallas import tpu as pltpu

_TB = 128


def _body(xs_ref, w1_ref, w2_ref, fw1_ref, fw2_ref, out_ref,
          x1, o1, p, o2, flat, h):
    TB = _TB
    f32 = jnp.float32
    bf16 = jnp.bfloat16

    ii = lax.broadcasted_iota(jnp.int32, (8, TB), 0)
    probe = ii.astype(f32)
    expect_fwd = jnp.where(ii == 0, 7, ii - 1).astype(f32)
    roll_is_fwd = (pltpu.roll(probe, 1, 0) == expect_fwd)[None]

    def roll_sub(v, shift):
        s = shift % 8
        fwd = pltpu.roll(v, s, 1)
        bwd = pltpu.roll(v, (8 - s) % 8, 1)
        return jnp.where(roll_is_fwd, fwd, bwd)

    ones_slab = jnp.ones((8, TB), bf16)

    for hh in range(28):
        cs = slice(hh * TB, (hh + 1) * TB)
        for di in range(3):
            for t in range(6):
                r0 = (di * 6 + t) * 8
                x1[r0:r0 + 8, cs] = xs_ref[t, hh + di]
        x1[144:152, cs] = ones_slab

    o1[...] = jnp.dot(w1_ref[...], x1[...], preferred_element_type=f32)

    p[:192, 0:TB] = jnp.zeros((192, TB), bf16)
    p[:192, 15 * TB:16 * TB] = jnp.zeros((192, TB), bf16)
    p[192:200, :] = jnp.ones((8, 16 * TB), bf16)

    jj = lax.broadcasted_iota(jnp.int32, (6, 8, TB), 1)
    mask7 = jj < 7

    for r in range(14):
        a = o1[:, (2 * r) * TB:(2 * r + 1) * TB]
        b = o1[:, (2 * r + 1) * TB:(2 * r + 2) * TB]
        m = jnp.maximum(a, b).reshape(6, 4, 8, TB)
        q0 = jnp.maximum(m[:, 0], m[:, 1])
        q1 = jnp.maximum(m[:, 2], m[:, 3])
        q0 = jnp.where(mask7, jnp.maximum(q0, 0.0), 0.0)
        q1 = jnp.where(mask7, jnp.maximum(q1, 0.0), 0.0)
        t0 = roll_sub(q1, 1)
        t3 = jnp.where(mask7, roll_sub(q0, -1), 0.0)
        cs = slice((r + 1) * TB, (r + 2) * TB)
        p[0:48, cs] = t0.reshape(48, TB).astype(bf16)
        p[48:96, cs] = q0.reshape(48, TB).astype(bf16)
        p[96:144, cs] = q1.reshape(48, TB).astype(bf16)
        p[144:192, cs] = t3.reshape(48, TB).astype(bf16)

    o2[...] = jnp.dot(w2_ref[0], p[:, 0:14 * TB], preferred_element_type=f32)
    o2[...] += jnp.dot(w2_ref[1], p[:, TB:15 * TB], preferred_element_type=f32)
    o2[...] += jnp.dot(w2_ref[2], p[:, 2 * TB:16 * TB], preferred_element_type=f32)

    for rp in range(7):
        a = o2[:, (2 * rp) * TB:(2 * rp + 1) * TB]
        b = o2[:, (2 * rp + 1) * TB:(2 * rp + 2) * TB]
        m = jnp.maximum(a, b).reshape(6, 2, 8, TB)
        f = jnp.maximum(jnp.maximum(m[:, 0], m[:, 1]), 0.0).astype(bf16)
        for c in range(6):
            r0 = c * 56 + rp * 8
            flat[r0:r0 + 8, :] = f[c]
    flat[336:344, :] = ones_slab

    hv = jnp.dot(fw1_ref[...], flat[...], preferred_element_type=f32)
    h[0:64, :] = jnp.maximum(hv, 0.0).astype(bf16)
    h[64:72, :] = ones_slab
    out_ref[...] = jnp.dot(fw2_ref[...], h[...], preferred_element_type=f32)


@jax.jit
def _forward(x, w1, b1, w2, b2, fw1, fb1, fw2, fb2):
    B = x.shape[0]
    TB = _TB
    n_tiles = pl.cdiv(B, TB)
    B_pad = n_tiles * TB
    f32 = jnp.float32
    bf16 = jnp.bfloat16

    xn = x[:, 0].astype(f32)
    if B_pad != B:
        xn = jnp.pad(xn, ((0, B_pad - B), (0, 0), (0, 0)))
    xpad = jnp.pad(xn, ((0, 0), (1, 1), (1, 7)))
    xs = jnp.stack([xpad[:, :, t:t + 32:4] for t in range(6)], axis=0)
    xs = jnp.transpose(xs, (0, 2, 3, 1)).astype(bf16)

    w1r = w1.reshape(6, 3, 3).astype(f32)
    A = jnp.zeros((6, 4, 3, 6), f32)
    for p_ in range(4):
        for dj in range(3):
            A = A.at[:, p_, :, p_ + dj].set(w1r[:, :, dj])
    eye8 = jnp.eye(8, dtype=f32)
    W1 = jnp.einsum('cpdt,jk->cpjdtk', A, eye8).reshape(192, 144)
    W1f = jnp.zeros((192, 152), f32).at[:, :144].set(W1)
    W1f = W1f.at[:, 144].set(jnp.repeat(b1.astype(f32), 32)).astype(bf16)

    Bm = jnp.zeros((3, 6, 2, 4, 6), f32)
    for di in range(3):
        for q in range(2):
            for dj in range(3):
                Bm = Bm.at[di, :, q, q + dj, :].set(w2[:, :, di, dj].astype(f32))
    W2 = jnp.einsum('dcqti,jk->dcqjtik', Bm, eye8).reshape(3, 96, 192)
    W2f = jnp.zeros((3, 96, 200), f32).at[:, :, :192].set(W2)
    W2f = W2f.at[1, :, 192].set(jnp.repeat(b2.astype(f32), 16)).astype(bf16)

    fw1r = fw1.reshape(64, 6, 7, 7).astype(f32)
    fw1p = jnp.pad(fw1r, ((0, 0), (0, 0), (0, 0), (0, 1))).reshape(64, 336)
    FW1 = jnp.zeros((64, 344), f32).at[:, :336].set(fw1p)
    FW1 = FW1.at[:, 336].set(fb1.astype(f32)).astype(bf16)
    FW2 = jnp.zeros((10, 72), f32).at[:, :64].set(fw2.astype(f32))
    FW2 = FW2.at[:, 64].set(fb2.astype(f32)).astype(bf16)

    out = pl.pallas_call(
        _body,
        out_shape=jax.ShapeDtypeStruct((10, B_pad), f32),
        grid=(n_tiles,),
        in_specs=[
            pl.BlockSpec((6, 30, 8, TB), lambda i: (0, 0, 0, i)),
            pl.BlockSpec((192, 152), lambda i: (0, 0)),
            pl.BlockSpec((3, 96, 200), lambda i: (0, 0, 0)),
            pl.BlockSpec((64, 344), lambda i: (0, 0)),
            pl.BlockSpec((10, 72), lambda i: (0, 0)),
        ],
        out_specs=pl.BlockSpec((10, TB), lambda i: (0, i)),
        scratch_shapes=[
            pltpu.VMEM((152, 28 * TB), bf16),
            pltpu.VMEM((192, 28 * TB), f32),
            pltpu.VMEM((200, 16 * TB), bf16),
            pltpu.VMEM((96, 14 * TB), f32),
            pltpu.VMEM((344, TB), bf16),
            pltpu.VMEM((72, TB), bf16),
        ],
        compiler_params=pltpu.CompilerParams(
            dimension_semantics=("parallel",),
            vmem_limit_bytes=32 * 1024 * 1024,
        ),
    )(xs, W1f, W2f, FW1, FW2)

    return jnp.transpose(out[:, :B])


def kernel(x, w1, b1, w2, b2, fw1, fb1, fw2, fb2):
    return _forward(x, w1, b1, w2, b2, fw1, fb1, fw2, fb2)
